# baseline (speedup 1.0000x reference)
import functools

import jax
import jax.numpy as jnp
from jax import lax
from jax.experimental import pallas as pl
from jax.experimental.pallas import tpu as pltpu
from jax.experimental.pallas import tpu_sc as plsc

NUM_UTTS = 16
NUM_PATHS = 3200
TOTAL_ARCS = 1638400

NC = 2
NS = 16
L = 16
NW = NC * NS
ARCS_PER_WORKER = TOTAL_ARCS // NW
VECS_PER_WORKER = ARCS_PER_WORKER // L


CHUNK = 12800
NCHUNKS = ARCS_PER_WORKER // CHUNK


def _sc_segment_sum_body(
    scores_hbm, ids_hbm, out_hbm, scores_v, ids_v, acc_v, *sems
):
    wid = lax.axis_index("s") * NC + lax.axis_index("c")
    base = wid * ARCS_PER_WORKER
    lane = jax.lax.iota(jnp.int32, L)
    zeros = jnp.zeros((L,), jnp.float32)
    is_last_lane = lane == (L - 1)
    not_last_lane = lane < (L - 1)

    def start(k):
        sl = pl.ds(base + k * CHUNK, CHUNK)
        return (
            pltpu.async_copy(scores_hbm.at[sl], scores_v.at[k], sems[k]),
            pltpu.async_copy(ids_hbm.at[sl], ids_v.at[k, pl.ds(0, CHUNK)], sems[k]),
        )

    pending = [start(k) for k in range(NCHUNKS)]

    with jax.named_scope("acc_zero"):

        @plsc.parallel_loop(0, NUM_PATHS // L, unroll=8)
        def zero_body(i):
            acc_v[pl.ds(i * L, L)] = zeros

    for k in range(NCHUNKS):
        b = k
        with jax.named_scope("dma_wait"):
            for h in pending[k]:
                h.wait()

        with jax.named_scope("scatter"):

            @plsc.parallel_loop(0, CHUNK // L, unroll=16)
            def scatter_body(i):
                idx = ids_v[b, pl.ds(i * L, L)]
                idn = ids_v[b, pl.ds(i * L + 1, L)]
                val = scores_v[b, pl.ds(i * L, L)]
                s = plsc.cumsum(val)
                bnd = jnp.not_equal(idx, idn)
                plsc.addupdate_scatter(acc_v, [idx], s, mask=bnd | is_last_lane)
                plsc.addupdate_scatter(acc_v, [idn], -s, mask=bnd & not_last_lane)

    pltpu.sync_copy(acc_v, out_hbm.at[wid])


@jax.jit
def _sc_segment_sum(arc_scores, arc_to_path):
    mesh = plsc.VectorSubcoreMesh(
        core_axis_name="c", subcore_axis_name="s", num_cores=NC, num_subcores=NS
    )
    return pl.kernel(
        _sc_segment_sum_body,
        out_type=jax.ShapeDtypeStruct((NW, NUM_PATHS), jnp.float32),
        mesh=mesh,
        scratch_types=[
            pltpu.VMEM((NCHUNKS, CHUNK), jnp.float32),
            pltpu.VMEM((NCHUNKS, CHUNK + L), jnp.int32),
            pltpu.VMEM((NUM_PATHS,), jnp.float32),
        ]
        + [pltpu.SemaphoreType.DMA] * NCHUNKS,
        compiler_params=pltpu.CompilerParams(needs_layout_passes=False),
        name="mwer_segment_sum_sc",
    )(arc_scores, arc_to_path)


def _tc_tail_body(partials_ref, wers_ref, utt_ref, out_ref):
    path_logp = jnp.sum(partials_ref[...], axis=0, keepdims=True)
    prob = jnp.exp(path_logp)
    pw = prob * wers_ref[...].reshape(1, NUM_PATHS)
    utt = utt_ref[...].reshape(1, NUM_PATHS)
    loss = jnp.float32(0.0)
    for u in range(NUM_UTTS):
        m = utt == u
        den = jnp.sum(jnp.where(m, prob, 0.0))
        num = jnp.sum(jnp.where(m, pw, 0.0))
        loss = loss + jnp.where(den > 0, num / den, 0.0)
    out_ref[...] = jnp.broadcast_to(loss, (1, 1))


@jax.jit
def _tc_tail(partials, wers, path_to_utt):
    return pl.pallas_call(
        _tc_tail_body,
        out_shape=jax.ShapeDtypeStruct((1, 1), jnp.float32),
    )(partials, wers, path_to_utt)


def kernel(arc_scores, wers, arc_to_path, path_to_utt):
    partials = _sc_segment_sum(arc_scores, arc_to_path)
    loss = _tc_tail(partials, wers, path_to_utt)
    return loss[0, 0]


# speedup vs baseline: 1.0527x; 1.0527x over previous
import functools

import jax
import jax.numpy as jnp
from jax import lax
from jax.experimental import pallas as pl
from jax.experimental.pallas import tpu as pltpu
from jax.experimental.pallas import tpu_sc as plsc

NUM_UTTS = 16
NUM_PATHS = 3200
TOTAL_ARCS = 1638400

NC = 2
NS = 16
L = 16
NW = NC * NS
ARCS_PER_WORKER = TOTAL_ARCS // NW
VECS_PER_WORKER = ARCS_PER_WORKER // L


CHUNK = 12800
NCHUNKS = ARCS_PER_WORKER // CHUNK


def _sc_segment_sum_body(
    scores_hbm, ids_hbm, out_hbm, scores_v, ids_v, acc_v, *sems
):
    wid = lax.axis_index("s") * NC + lax.axis_index("c")
    base = wid * ARCS_PER_WORKER
    lane = jax.lax.iota(jnp.int32, L)
    zeros = jnp.zeros((L,), jnp.float32)
    is_last_lane = lane == (L - 1)
    not_last_lane = lane < (L - 1)

    def start(k):
        sl = pl.ds(base + k * CHUNK, CHUNK)
        return (
            pltpu.async_copy(scores_hbm.at[sl], scores_v.at[k], sems[k]),
            pltpu.async_copy(ids_hbm.at[sl], ids_v.at[k, pl.ds(0, CHUNK)], sems[k]),
        )

    pending = [start(k) for k in range(NCHUNKS)]

    with jax.named_scope("acc_zero"):

        @plsc.parallel_loop(0, NUM_PATHS // L, unroll=8)
        def zero_body(i):
            acc_v[pl.ds(i * L, L)] = zeros

    for k in range(NCHUNKS):
        b = k
        with jax.named_scope("dma_wait"):
            for h in pending[k]:
                h.wait()

        with jax.named_scope("scatter"):

            @plsc.parallel_loop(0, CHUNK // L, unroll=8)
            def scatter_body(i):
                idx = ids_v[b, pl.ds(i * L, L)]
                idn = ids_v[b, pl.ds(i * L + 1, L)]
                val = scores_v[b, pl.ds(i * L, L)]
                s = plsc.cumsum(val)
                bnd = jnp.not_equal(idx, idn)
                plsc.addupdate_scatter(acc_v, [idx], s, mask=bnd | is_last_lane)
                plsc.addupdate_scatter(acc_v, [idn], -s, mask=bnd & not_last_lane)

    pltpu.sync_copy(acc_v, out_hbm.at[wid])


@jax.jit
def _sc_segment_sum(arc_scores, arc_to_path):
    mesh = plsc.VectorSubcoreMesh(
        core_axis_name="c", subcore_axis_name="s", num_cores=NC, num_subcores=NS
    )
    return pl.kernel(
        _sc_segment_sum_body,
        out_type=jax.ShapeDtypeStruct((NW, NUM_PATHS), jnp.float32),
        mesh=mesh,
        scratch_types=[
            pltpu.VMEM((NCHUNKS, CHUNK), jnp.float32),
            pltpu.VMEM((NCHUNKS, CHUNK + L), jnp.int32),
            pltpu.VMEM((NUM_PATHS,), jnp.float32),
        ]
        + [pltpu.SemaphoreType.DMA] * NCHUNKS,
        compiler_params=pltpu.CompilerParams(needs_layout_passes=False),
        name="mwer_segment_sum_sc",
    )(arc_scores, arc_to_path)


def _tc_tail_body(partials_ref, wers_ref, utt_ref, out_ref):
    path_logp = jnp.sum(partials_ref[...], axis=0, keepdims=True)
    prob = jnp.exp(path_logp)
    pw = prob * wers_ref[...].reshape(1, NUM_PATHS)
    utt = utt_ref[...].reshape(1, NUM_PATHS)
    loss = jnp.float32(0.0)
    for u in range(NUM_UTTS):
        m = utt == u
        den = jnp.sum(jnp.where(m, prob, 0.0))
        num = jnp.sum(jnp.where(m, pw, 0.0))
        loss = loss + jnp.where(den > 0, num / den, 0.0)
    out_ref[...] = jnp.broadcast_to(loss, (1, 1))


@jax.jit
def _tc_tail(partials, wers, path_to_utt):
    return pl.pallas_call(
        _tc_tail_body,
        out_shape=jax.ShapeDtypeStruct((1, 1), jnp.float32),
    )(partials, wers, path_to_utt)


def kernel(arc_scores, wers, arc_to_path, path_to_utt):
    partials = _sc_segment_sum(arc_scores, arc_to_path)
    loss = _tc_tail(partials, wers, path_to_utt)
    return loss[0, 0]


# speedup vs baseline: 1.0779x; 1.0239x over previous
import functools

import jax
import jax.numpy as jnp
from jax import lax
from jax.experimental import pallas as pl
from jax.experimental.pallas import tpu as pltpu
from jax.experimental.pallas import tpu_sc as plsc

NUM_UTTS = 16
NUM_PATHS = 3200
TOTAL_ARCS = 1638400

NC = 2
NS = 16
L = 16
NW = NC * NS
ARCS_PER_WORKER = TOTAL_ARCS // NW
VECS_PER_WORKER = ARCS_PER_WORKER // L


CHUNK = 6400
NCHUNKS = ARCS_PER_WORKER // CHUNK


def _sc_segment_sum_body(
    scores_hbm, ids_hbm, out_hbm, scores_v, ids_v, acc_v, *sems
):
    wid = lax.axis_index("s") * NC + lax.axis_index("c")
    base = wid * ARCS_PER_WORKER
    lane = jax.lax.iota(jnp.int32, L)
    zeros = jnp.zeros((L,), jnp.float32)
    is_last_lane = lane == (L - 1)
    not_last_lane = lane < (L - 1)
    shift_up = jnp.minimum(lane + 1, L - 1)

    def start(k):
        sl = pl.ds(base + k * CHUNK, CHUNK)
        return (
            pltpu.async_copy(scores_hbm.at[sl], scores_v.at[k], sems[k]),
            pltpu.async_copy(ids_hbm.at[sl], ids_v.at[k, pl.ds(0, CHUNK)], sems[k]),
        )

    pending = [start(k) for k in range(NCHUNKS)]

    with jax.named_scope("acc_zero"):

        @plsc.parallel_loop(0, NUM_PATHS // L, unroll=8)
        def zero_body(i):
            acc_v[pl.ds(i * L, L)] = zeros

    for k in range(NCHUNKS):
        b = k
        with jax.named_scope("dma_wait"):
            for h in pending[k]:
                h.wait()

        with jax.named_scope("scatter"):

            @plsc.parallel_loop(0, CHUNK // L, unroll=8)
            def scatter_body(i):
                idx = ids_v[b, pl.ds(i * L, L)]
                val = scores_v[b, pl.ds(i * L, L)]
                idn = jnp.take_along_axis(
                    idx, shift_up, axis=0,
                    mode=lax.GatherScatterMode.PROMISE_IN_BOUNDS,
                )
                s = plsc.cumsum(val)
                bnd = jnp.not_equal(idx, idn)
                plsc.addupdate_scatter(acc_v, [idx], s, mask=bnd | is_last_lane)
                plsc.addupdate_scatter(acc_v, [idn], -s, mask=bnd & not_last_lane)

    pltpu.sync_copy(acc_v, out_hbm.at[wid])


@jax.jit
def _sc_segment_sum(arc_scores, arc_to_path):
    mesh = plsc.VectorSubcoreMesh(
        core_axis_name="c", subcore_axis_name="s", num_cores=NC, num_subcores=NS
    )
    return pl.kernel(
        _sc_segment_sum_body,
        out_type=jax.ShapeDtypeStruct((NW, NUM_PATHS), jnp.float32),
        mesh=mesh,
        scratch_types=[
            pltpu.VMEM((NCHUNKS, CHUNK), jnp.float32),
            pltpu.VMEM((NCHUNKS, CHUNK + L), jnp.int32),
            pltpu.VMEM((NUM_PATHS,), jnp.float32),
        ]
        + [pltpu.SemaphoreType.DMA] * NCHUNKS,
        compiler_params=pltpu.CompilerParams(needs_layout_passes=False),
        name="mwer_segment_sum_sc",
    )(arc_scores, arc_to_path)


def _tc_tail_body(partials_ref, wers_ref, utt_ref, out_ref):
    path_logp = jnp.sum(partials_ref[...], axis=0, keepdims=True)
    prob = jnp.exp(path_logp)
    pw = prob * wers_ref[...].reshape(1, NUM_PATHS)
    utt = utt_ref[...].reshape(1, NUM_PATHS)
    loss = jnp.float32(0.0)
    for u in range(NUM_UTTS):
        m = utt == u
        den = jnp.sum(jnp.where(m, prob, 0.0))
        num = jnp.sum(jnp.where(m, pw, 0.0))
        loss = loss + jnp.where(den > 0, num / den, 0.0)
    out_ref[...] = jnp.broadcast_to(loss, (1, 1))


@jax.jit
def _tc_tail(partials, wers, path_to_utt):
    return pl.pallas_call(
        _tc_tail_body,
        out_shape=jax.ShapeDtypeStruct((1, 1), jnp.float32),
    )(partials, wers, path_to_utt)


def kernel(arc_scores, wers, arc_to_path, path_to_utt):
    partials = _sc_segment_sum(arc_scores, arc_to_path)
    loss = _tc_tail(partials, wers, path_to_utt)
    return loss[0, 0]


# speedup vs baseline: 1.0844x; 1.0061x over previous
import functools

import jax
import jax.numpy as jnp
from jax import lax
from jax.experimental import pallas as pl
from jax.experimental.pallas import tpu as pltpu
from jax.experimental.pallas import tpu_sc as plsc

NUM_UTTS = 16
NUM_PATHS = 3200
TOTAL_ARCS = 1638400

NC = 2
NS = 16
L = 16
NW = NC * NS
ARCS_PER_WORKER = TOTAL_ARCS // NW
VECS_PER_WORKER = ARCS_PER_WORKER // L


CHUNK = 6400
NCHUNKS = ARCS_PER_WORKER // CHUNK


def _sc_segment_sum_body(
    scores_hbm, ids_hbm, out_hbm, scores_v, ids_v, acc_v, *sems
):
    wid = lax.axis_index("s") * NC + lax.axis_index("c")
    base = wid * ARCS_PER_WORKER
    lane = jax.lax.iota(jnp.int32, L)
    zeros = jnp.zeros((L,), jnp.float32)
    is_last_lane = lane == (L - 1)
    not_last_lane = lane < (L - 1)
    shift_up = jnp.minimum(lane + 1, L - 1)

    def start(k):
        sl = pl.ds(base + k * CHUNK, CHUNK)
        return (
            pltpu.async_copy(scores_hbm.at[sl], scores_v.at[k], sems[k]),
            pltpu.async_copy(ids_hbm.at[sl], ids_v.at[k, pl.ds(0, CHUNK)], sems[k]),
        )

    pending = [start(k) for k in range(NCHUNKS)]

    with jax.named_scope("acc_zero"):

        @plsc.parallel_loop(0, NUM_PATHS // L, unroll=4)
        def zero_body(i):
            acc_v[pl.ds(i * L, L)] = zeros

    for k in range(NCHUNKS):
        b = k
        with jax.named_scope("dma_wait"):
            for h in pending[k]:
                h.wait()

        with jax.named_scope("scatter"):

            @plsc.parallel_loop(0, CHUNK // L, unroll=4)
            def scatter_body(i):
                idx = ids_v[b, pl.ds(i * L, L)]
                val = scores_v[b, pl.ds(i * L, L)]
                idn = jnp.take_along_axis(
                    idx, shift_up, axis=0,
                    mode=lax.GatherScatterMode.PROMISE_IN_BOUNDS,
                )
                s = plsc.cumsum(val)
                bnd = jnp.not_equal(idx, idn)
                plsc.addupdate_scatter(acc_v, [idx], s, mask=bnd | is_last_lane)
                plsc.addupdate_scatter(acc_v, [idn], -s, mask=bnd & not_last_lane)

    pltpu.sync_copy(acc_v, out_hbm.at[wid])


@jax.jit
def _sc_segment_sum(arc_scores, arc_to_path):
    mesh = plsc.VectorSubcoreMesh(
        core_axis_name="c", subcore_axis_name="s", num_cores=NC, num_subcores=NS
    )
    return pl.kernel(
        _sc_segment_sum_body,
        out_type=jax.ShapeDtypeStruct((NW, NUM_PATHS), jnp.float32),
        mesh=mesh,
        scratch_types=[
            pltpu.VMEM((NCHUNKS, CHUNK), jnp.float32),
            pltpu.VMEM((NCHUNKS, CHUNK + L), jnp.int32),
            pltpu.VMEM((NUM_PATHS,), jnp.float32),
        ]
        + [pltpu.SemaphoreType.DMA] * NCHUNKS,
        compiler_params=pltpu.CompilerParams(needs_layout_passes=False),
        name="mwer_segment_sum_sc",
    )(arc_scores, arc_to_path)


def _tc_tail_body(partials_ref, wers_ref, utt_ref, out_ref):
    path_logp = jnp.sum(partials_ref[...], axis=0, keepdims=True)
    prob = jnp.exp(path_logp)
    pw = prob * wers_ref[...].reshape(1, NUM_PATHS)
    utt = utt_ref[...].reshape(1, NUM_PATHS)
    loss = jnp.float32(0.0)
    for u in range(NUM_UTTS):
        m = utt == u
        den = jnp.sum(jnp.where(m, prob, 0.0))
        num = jnp.sum(jnp.where(m, pw, 0.0))
        loss = loss + jnp.where(den > 0, num / den, 0.0)
    out_ref[...] = jnp.broadcast_to(loss, (1, 1))


@jax.jit
def _tc_tail(partials, wers, path_to_utt):
    return pl.pallas_call(
        _tc_tail_body,
        out_shape=jax.ShapeDtypeStruct((1, 1), jnp.float32),
    )(partials, wers, path_to_utt)


def kernel(arc_scores, wers, arc_to_path, path_to_utt):
    partials = _sc_segment_sum(arc_scores, arc_to_path)
    loss = _tc_tail(partials, wers, path_to_utt)
    return loss[0, 0]


# speedup vs baseline: 1.1067x; 1.0205x over previous
import functools

import jax
import jax.numpy as jnp
from jax import lax
from jax.experimental import pallas as pl
from jax.experimental.pallas import tpu as pltpu
from jax.experimental.pallas import tpu_sc as plsc

NUM_UTTS = 16
NUM_PATHS = 3200
TOTAL_ARCS = 1638400

NC = 2
NS = 16
L = 16
NW = NC * NS
ARCS_PER_WORKER = TOTAL_ARCS // NW
VECS_PER_WORKER = ARCS_PER_WORKER // L


CHUNK = 12800
NCHUNKS = ARCS_PER_WORKER // CHUNK


def _sc_segment_sum_body(
    scores_hbm, ids_hbm, out_hbm, scores_v, ids_v, acc_v, *sems
):
    wid = lax.axis_index("s") * NC + lax.axis_index("c")
    base = wid * ARCS_PER_WORKER
    lane = jax.lax.iota(jnp.int32, L)
    zeros = jnp.zeros((L,), jnp.float32)
    is_last_lane = lane == (L - 1)
    not_last_lane = lane < (L - 1)
    shift_up = jnp.minimum(lane + 1, L - 1)

    def start(k):
        sl = pl.ds(base + k * CHUNK, CHUNK)
        return (
            pltpu.async_copy(scores_hbm.at[sl], scores_v.at[k], sems[k]),
            pltpu.async_copy(ids_hbm.at[sl], ids_v.at[k, pl.ds(0, CHUNK)], sems[k]),
        )

    pending = [start(k) for k in range(NCHUNKS)]

    with jax.named_scope("acc_zero"):

        @plsc.parallel_loop(0, NUM_PATHS // L, unroll=4)
        def zero_body(i):
            acc_v[pl.ds(i * L, L)] = zeros

    for k in range(NCHUNKS):
        b = k
        with jax.named_scope("dma_wait"):
            for h in pending[k]:
                h.wait()

        with jax.named_scope("scatter"):

            @plsc.parallel_loop(0, CHUNK // L, unroll=4)
            def scatter_body(i):
                idx = ids_v[b, pl.ds(i * L, L)]
                val = scores_v[b, pl.ds(i * L, L)]
                idn = jnp.take_along_axis(
                    idx, shift_up, axis=0,
                    mode=lax.GatherScatterMode.PROMISE_IN_BOUNDS,
                )
                s = plsc.cumsum(val)
                bnd = jnp.not_equal(idx, idn)
                plsc.addupdate_scatter(acc_v, [idx], s, mask=bnd | is_last_lane)
                plsc.addupdate_scatter(acc_v, [idn], -s, mask=bnd)

    pltpu.sync_copy(acc_v, out_hbm.at[wid])


@jax.jit
def _sc_segment_sum(arc_scores, arc_to_path):
    mesh = plsc.VectorSubcoreMesh(
        core_axis_name="c", subcore_axis_name="s", num_cores=NC, num_subcores=NS
    )
    return pl.kernel(
        _sc_segment_sum_body,
        out_type=jax.ShapeDtypeStruct((NW, NUM_PATHS), jnp.float32),
        mesh=mesh,
        scratch_types=[
            pltpu.VMEM((NCHUNKS, CHUNK), jnp.float32),
            pltpu.VMEM((NCHUNKS, CHUNK + L), jnp.int32),
            pltpu.VMEM((NUM_PATHS,), jnp.float32),
        ]
        + [pltpu.SemaphoreType.DMA] * NCHUNKS,
        compiler_params=pltpu.CompilerParams(needs_layout_passes=False),
        name="mwer_segment_sum_sc",
    )(arc_scores, arc_to_path)


def _tc_tail_body(partials_ref, wers_ref, utt_ref, out_ref):
    path_logp = jnp.sum(partials_ref[...], axis=0, keepdims=True)
    prob = jnp.exp(path_logp)
    pw = prob * wers_ref[...].reshape(1, NUM_PATHS)
    utt = utt_ref[...].reshape(1, NUM_PATHS)
    loss = jnp.float32(0.0)
    for u in range(NUM_UTTS):
        m = utt == u
        den = jnp.sum(jnp.where(m, prob, 0.0))
        num = jnp.sum(jnp.where(m, pw, 0.0))
        loss = loss + jnp.where(den > 0, num / den, 0.0)
    out_ref[...] = jnp.broadcast_to(loss, (1, 1))


@jax.jit
def _tc_tail(partials, wers, path_to_utt):
    return pl.pallas_call(
        _tc_tail_body,
        out_shape=jax.ShapeDtypeStruct((1, 1), jnp.float32),
    )(partials, wers, path_to_utt)


def kernel(arc_scores, wers, arc_to_path, path_to_utt):
    partials = _sc_segment_sum(arc_scores, arc_to_path)
    loss = _tc_tail(partials, wers, path_to_utt)
    return loss[0, 0]


# speedup vs baseline: 1.1127x; 1.0054x over previous
import functools

import jax
import jax.numpy as jnp
from jax import lax
from jax.experimental import pallas as pl
from jax.experimental.pallas import tpu as pltpu
from jax.experimental.pallas import tpu_sc as plsc

NUM_UTTS = 16
NUM_PATHS = 3200
TOTAL_ARCS = 1638400

NC = 2
NS = 16
L = 16
NW = NC * NS
ARCS_PER_WORKER = TOTAL_ARCS // NW
VECS_PER_WORKER = ARCS_PER_WORKER // L


CHUNK_SIZES = (3200, 16000, 16000, 16000)
CHUNK_OFFS = (0, 3200, 19200, 35200)
NCHUNKS = len(CHUNK_SIZES)


def _sc_segment_sum_body(
    scores_hbm, ids_hbm, out_hbm, scores_v, ids_v, acc_v, *sems
):
    wid = lax.axis_index("s") * NC + lax.axis_index("c")
    base = wid * ARCS_PER_WORKER
    lane = jax.lax.iota(jnp.int32, L)
    zeros = jnp.zeros((L,), jnp.float32)
    is_last_lane = lane == (L - 1)
    not_last_lane = lane < (L - 1)
    shift_up = jnp.minimum(lane + 1, L - 1)

    def start(k):
        off, n = CHUNK_OFFS[k], CHUNK_SIZES[k]
        sl = pl.ds(base + off, n)
        dst = pl.ds(off, n)
        return (
            pltpu.async_copy(scores_hbm.at[sl], scores_v.at[dst], sems[k]),
            pltpu.async_copy(ids_hbm.at[sl], ids_v.at[dst], sems[k]),
        )

    pending = [start(k) for k in range(NCHUNKS)]

    with jax.named_scope("acc_zero"):

        @plsc.parallel_loop(0, NUM_PATHS // L, unroll=4)
        def zero_body(i):
            acc_v[pl.ds(i * L, L)] = zeros

    for k in range(NCHUNKS):
        v0 = CHUNK_OFFS[k] // L
        with jax.named_scope("dma_wait"):
            for h in pending[k]:
                h.wait()

        with jax.named_scope("scatter"):

            @plsc.parallel_loop(v0, v0 + CHUNK_SIZES[k] // L, unroll=4)
            def scatter_body(i):
                idx = ids_v[pl.ds(i * L, L)]
                val = scores_v[pl.ds(i * L, L)]
                idn = jnp.take_along_axis(
                    idx, shift_up, axis=0,
                    mode=lax.GatherScatterMode.PROMISE_IN_BOUNDS,
                )
                s = plsc.cumsum(val)
                bnd = jnp.not_equal(idx, idn)
                plsc.addupdate_scatter(acc_v, [idx], s, mask=bnd | is_last_lane)
                plsc.addupdate_scatter(acc_v, [idn], -s, mask=bnd)

    pltpu.sync_copy(acc_v, out_hbm.at[wid])


@jax.jit
def _sc_segment_sum(arc_scores, arc_to_path):
    mesh = plsc.VectorSubcoreMesh(
        core_axis_name="c", subcore_axis_name="s", num_cores=NC, num_subcores=NS
    )
    return pl.kernel(
        _sc_segment_sum_body,
        out_type=jax.ShapeDtypeStruct((NW, NUM_PATHS), jnp.float32),
        mesh=mesh,
        scratch_types=[
            pltpu.VMEM((ARCS_PER_WORKER,), jnp.float32),
            pltpu.VMEM((ARCS_PER_WORKER,), jnp.int32),
            pltpu.VMEM((NUM_PATHS,), jnp.float32),
        ]
        + [pltpu.SemaphoreType.DMA] * NCHUNKS,
        compiler_params=pltpu.CompilerParams(needs_layout_passes=False),
        name="mwer_segment_sum_sc",
    )(arc_scores, arc_to_path)


def _tc_tail_body(partials_ref, wers_ref, utt_ref, out_ref):
    path_logp = jnp.sum(partials_ref[...], axis=0, keepdims=True)
    prob = jnp.exp(path_logp)
    pw = prob * wers_ref[...].reshape(1, NUM_PATHS)
    utt = utt_ref[...].reshape(1, NUM_PATHS)
    loss = jnp.float32(0.0)
    for u in range(NUM_UTTS):
        m = utt == u
        den = jnp.sum(jnp.where(m, prob, 0.0))
        num = jnp.sum(jnp.where(m, pw, 0.0))
        loss = loss + jnp.where(den > 0, num / den, 0.0)
    out_ref[...] = jnp.broadcast_to(loss, (1, 1))


@jax.jit
def _tc_tail(partials, wers, path_to_utt):
    return pl.pallas_call(
        _tc_tail_body,
        out_shape=jax.ShapeDtypeStruct((1, 1), jnp.float32),
    )(partials, wers, path_to_utt)


def kernel(arc_scores, wers, arc_to_path, path_to_utt):
    partials = _sc_segment_sum(arc_scores, arc_to_path)
    loss = _tc_tail(partials, wers, path_to_utt)
    return loss[0, 0]
